# baseline (speedup 1.0000x reference)
import functools
import math

import jax
import jax.numpy as jnp
from jax import lax
from jax.experimental import pallas as pl
from jax.experimental.pallas import tpu as pltpu
from jax.experimental.pallas import tpu_sc as plsc

_K = 0.05
_K_PERCENT = 0.5

_NC, _NS, _L = 2, 16, 16
_NW = _NC * _NS


def _sc_body(k_row, rows_per_w, cap_row, x_hbm, thr_hbm, svk_hbm,
             row_v, key_v, hist_v, scum_v, thr_v, svk_v):
    n = x_hbm.shape[1]
    chunks = n // _L
    wid = lax.axis_index("s") * _NC + lax.axis_index("c")
    lane = lax.iota(jnp.int32, _L)
    ones_i = jnp.ones((_L,), jnp.int32)
    zeros_i = jnp.zeros((_L,), jnp.int32)
    neginf = jnp.full((_L,), -jnp.inf, jnp.float32)

    def _init(i, c):
        svk_v[pl.ds(i * _L, _L)] = neginf
        return c

    lax.fori_loop(0, (rows_per_w * cap_row) // _L, _init, 0)
    scum_v[pl.ds(256, _L)] = zeros_i

    def row_body(j, carry):
        r = wid * rows_per_w + j
        pltpu.sync_copy(x_hbm.at[r], row_v)

        def _keys(i, c):
            v = row_v[pl.ds(i * _L, _L)]
            s = lax.bitcast_convert_type(v, jnp.int32)
            key = jnp.where(s >= 0, s ^ jnp.int32(-(2**31)), ~s)
            key_v[pl.ds(i * _L, _L)] = key
            return c

        lax.fori_loop(0, chunks, _keys, 0)

        prefix = jnp.int32(0)
        rank = jnp.int32(k_row)
        for t in range(4):
            shift = 24 - 8 * t

            def _zero(i, c):
                hist_v[pl.ds(i * _L, _L)] = zeros_i
                return c

            lax.fori_loop(0, (16 * 256) // _L, _zero, 0)

            if t == 0:
                def _sweep(i, c):
                    key = key_v[pl.ds(i * _L, _L)]
                    digit = lax.shift_right_logical(key, 24) & jnp.int32(0xFF)
                    plsc.addupdate_scatter(hist_v, [lane * 256 + digit], ones_i)
                    return c
            else:
                hi_shift = 32 - 8 * t
                want = lax.shift_right_logical(prefix, hi_shift)

                def _sweep(i, c):
                    key = key_v[pl.ds(i * _L, _L)]
                    digit = lax.shift_right_logical(key, shift) & jnp.int32(0xFF)
                    match = lax.shift_right_logical(key, hi_shift) == want
                    plsc.addupdate_scatter(hist_v, [lane * 256 + digit], ones_i,
                                           mask=match)
                    return c

            lax.fori_loop(0, chunks, _sweep, 0)

            def _chunk(i, cr):
                c = 15 - i

                def _lanesum(l, acc):
                    return acc + hist_v[pl.ds(l * 256 + c * _L, _L)]

                tot = lax.fori_loop(0, 16, _lanesum, zeros_i)
                svec = lax.rev(plsc.cumsum(lax.rev(tot, (0,))), (0,)) + cr
                scum_v[pl.ds(c * _L, _L)] = svec
                return jnp.max(svec)

            lax.fori_loop(0, 16, _chunk, jnp.int32(0))

            def _cnt(i, acc):
                svec = scum_v[pl.ds(i * _L, _L)]
                return acc + jnp.sum((svec >= rank).astype(jnp.int32))

            m = lax.fori_loop(0, 16, _cnt, jnp.int32(0))
            sel = m - 1
            cnt_above = scum_v[pl.ds(sel + 1, _L)][0]
            rank = rank - cnt_above
            prefix = prefix | lax.shift_left(sel, shift)

        tkey = jnp.full((_L,), prefix ^ jnp.int32(-(2**31)), jnp.int32)
        bits = jnp.where(tkey >= 0, tkey, tkey ^ jnp.int32(0x7FFFFFFF))
        thr_vec = lax.bitcast_convert_type(bits, jnp.float32)
        thr_v[pl.ds(j * _L, _L)] = thr_vec

        base0 = j * cap_row

        def _surv(i, cnt):
            v = row_v[pl.ds(i * _L, _L)]
            mask = (v >= thr_vec) & (v != 0.0)
            base = jnp.minimum(base0 + cnt, base0 + cap_row - _L)
            plsc.store_compressed(svk_v.at[pl.ds(base, _L)], v, mask=mask)
            return cnt + jnp.max(plsc.all_reduce_population_count(mask))

        lax.fori_loop(0, chunks, _surv, jnp.int32(0))
        return carry

    lax.fori_loop(0, rows_per_w, row_body, 0)

    pltpu.sync_copy(thr_v, thr_hbm.at[wid])
    pltpu.sync_copy(svk_v, svk_hbm.at[wid])


def _key_to_float(c):
    bits = jnp.where(c >= 0, c, c ^ jnp.int32(0x7FFFFFFF))
    return lax.bitcast_convert_type(bits, jnp.float32)


def _tc_finish(x_ref, thr_ref, svk_ref, out_ref, *, k_glob):
    sv = svk_ref[...]

    def count_ge(fc):
        return jnp.sum((sv >= fc).astype(jnp.int32))

    g0 = count_ge(jnp.float32(0.0))
    ans = jnp.where(g0 >= k_glob, jnp.int32(0), jnp.int32(-(2**31)))

    def body(i, ans):
        bit = jnp.int32(2**30) >> i
        cand = ans | bit
        cnt = count_ge(_key_to_float(cand))
        return jnp.where(cnt >= k_glob, cand, ans)

    ans = lax.fori_loop(0, 31, body, ans)
    tg = _key_to_float(ans)

    x = x_ref[...]
    thr = thr_ref[...]
    out_ref[...] = jnp.where((x >= thr) & (x != 0.0) & (x >= tg), x, 0.0)


def kernel(x):
    b, n = x.shape
    k_row = math.ceil(_K * n)
    k_glob = math.ceil(_K_PERCENT * b * k_row)
    rows_per_w = b // _NW
    cap_row = ((k_row + 25 + _L - 1) // _L) * _L

    mesh = plsc.VectorSubcoreMesh(core_axis_name="c", subcore_axis_name="s")
    sc = pl.kernel(
        functools.partial(_sc_body, k_row, rows_per_w, cap_row),
        out_type=(
            jax.ShapeDtypeStruct((_NW, rows_per_w * _L), jnp.float32),
            jax.ShapeDtypeStruct((_NW, rows_per_w * cap_row), jnp.float32),
        ),
        mesh=mesh,
        compiler_params=pltpu.CompilerParams(needs_layout_passes=False),
        scratch_types=[
            pltpu.VMEM((n,), jnp.float32),
            pltpu.VMEM((n,), jnp.int32),
            pltpu.VMEM((16 * 256,), jnp.int32),
            pltpu.VMEM((256 + _L,), jnp.int32),
            pltpu.VMEM((rows_per_w * _L,), jnp.float32),
            pltpu.VMEM((rows_per_w * cap_row,), jnp.float32),
        ],
    )
    thr_out, svk_out = sc(x)
    thr = thr_out.reshape(_NW, rows_per_w, _L)[:, :, 0].reshape(b, 1)

    return pl.pallas_call(
        functools.partial(_tc_finish, k_glob=k_glob),
        out_shape=jax.ShapeDtypeStruct((b, n), x.dtype),
        in_specs=[
            pl.BlockSpec(memory_space=pltpu.VMEM),
            pl.BlockSpec(memory_space=pltpu.VMEM),
            pl.BlockSpec(memory_space=pltpu.VMEM),
        ],
        out_specs=pl.BlockSpec(memory_space=pltpu.VMEM),
    )(x, thr, svk_out)


# speedup vs baseline: 3.1804x; 3.1804x over previous
import functools
import math

import jax
import jax.numpy as jnp
from jax import lax
from jax.experimental import pallas as pl
from jax.experimental.pallas import tpu as pltpu
from jax.experimental.pallas import tpu_sc as plsc

_K = 0.05
_K_PERCENT = 0.5

_NC, _NS, _L = 2, 16, 16
_NW = _NC * _NS


def _sc_body(k_row, rows_per_w, cap_row, x_hbm, thr_hbm, svk_hbm,
             row_v, pc_v, hist_v, scum_v, thr_v, svk_v):
    n = x_hbm.shape[1]
    chunks = n // _L
    wid = lax.axis_index("s") * _NC + lax.axis_index("c")
    lane = lax.iota(jnp.int32, _L)
    ones_i = jnp.ones((_L,), jnp.int32)
    zeros_i = jnp.zeros((_L,), jnp.int32)
    neginf = jnp.full((_L,), -jnp.inf, jnp.float32)

    def _ukey(v):
        s = lax.bitcast_convert_type(v, jnp.int32)
        return s ^ (lax.shift_right_arithmetic(s, 31) | jnp.int32(-(2**31)))

    @plsc.parallel_loop(0, (rows_per_w * cap_row) // _L, unroll=8)
    def _init(i):
        svk_v[pl.ds(i * _L, _L)] = neginf

    scum_v[pl.ds(256, _L)] = zeros_i

    def row_body(j, carry):
        r = wid * rows_per_w + j
        pltpu.sync_copy(x_hbm.at[r], row_v)

        prefix = jnp.int32(0)
        rank = jnp.int32(k_row)
        for t in range(4):
            shift = 24 - 8 * t

            @plsc.parallel_loop(0, (16 * 256) // _L, unroll=8)
            def _zero(i):
                hist_v[pl.ds(i * _L, _L)] = zeros_i

            if t == 0:
                @plsc.parallel_loop(0, chunks, unroll=8)
                def _sweep(i):
                    key = _ukey(row_v[pl.ds(i * _L, _L)])
                    digit = lax.shift_right_logical(key, 24) & jnp.int32(0xFF)
                    plsc.addupdate_scatter(hist_v, [lane * 256 + digit], ones_i)
            else:
                hi_shift = 32 - 8 * t
                want = lax.shift_right_logical(prefix, hi_shift)

                @plsc.parallel_loop(0, chunks, unroll=8)
                def _sweep(i):
                    key = _ukey(row_v[pl.ds(i * _L, _L)])
                    digit = lax.shift_right_logical(key, shift) & jnp.int32(0xFF)
                    match = lax.shift_right_logical(key, hi_shift) == want
                    plsc.addupdate_scatter(hist_v, [lane * 256 + digit], ones_i,
                                           mask=match)

            def _chunk(i, cr):
                c = 15 - i

                def _lanesum(l, acc):
                    return acc + hist_v[pl.ds(l * 256 + c * _L, _L)]

                tot = lax.fori_loop(0, 16, _lanesum, zeros_i)
                svec = lax.rev(plsc.cumsum(lax.rev(tot, (0,))), (0,)) + cr
                scum_v[pl.ds(c * _L, _L)] = svec
                return jnp.max(svec)

            lax.fori_loop(0, 16, _chunk, jnp.int32(0))

            def _cnt(i, acc):
                svec = scum_v[pl.ds(i * _L, _L)]
                return acc + jnp.sum((svec >= rank).astype(jnp.int32))

            m = lax.fori_loop(0, 16, _cnt, jnp.int32(0))
            sel = m - 1
            cnt_above = scum_v[pl.ds(sel + 1, _L)][0]
            rank = rank - cnt_above
            prefix = prefix | lax.shift_left(sel, shift)

        tkey = jnp.full((_L,), prefix ^ jnp.int32(-(2**31)), jnp.int32)
        bits = jnp.where(tkey >= 0, tkey, tkey ^ jnp.int32(0x7FFFFFFF))
        thr_vec = lax.bitcast_convert_type(bits, jnp.float32)
        thr_v[pl.ds(j * _L, _L)] = thr_vec

        base0 = j * cap_row
        lane0 = lane == 0

        @plsc.parallel_loop(0, chunks, unroll=8)
        def _pass_a(i):
            v = row_v[pl.ds(i * _L, _L)]
            mask = (v >= thr_vec) & (v != 0.0)
            pc = plsc.all_reduce_population_count(mask)
            plsc.store_scatter(pc_v, [jnp.full((_L,), i, jnp.int32)], pc,
                               mask=lane0)

        def _pass_b(g, run):
            vec = pc_v[pl.ds(g * _L, _L)]
            cs = plsc.cumsum(vec)
            pc_v[pl.ds(g * _L, _L)] = cs - vec + run
            return run + cs[_L - 1]

        lax.fori_loop(0, chunks // _L, _pass_b, jnp.int32(0))

        @plsc.parallel_loop(0, chunks, unroll=8)
        def _pass_c(i):
            v = row_v[pl.ds(i * _L, _L)]
            mask = (v >= thr_vec) & (v != 0.0)
            start = pc_v[pl.ds(i, _L)][0]
            base = base0 + jnp.minimum(start, jnp.int32(cap_row - _L))
            plsc.store_compressed(svk_v.at[pl.ds(base, _L)], v, mask=mask)

        return carry

    lax.fori_loop(0, rows_per_w, row_body, 0)

    pltpu.sync_copy(thr_v, thr_hbm.at[wid])
    pltpu.sync_copy(svk_v, svk_hbm.at[wid])


def _key_to_float(c):
    bits = jnp.where(c >= 0, c, c ^ jnp.int32(0x7FFFFFFF))
    return lax.bitcast_convert_type(bits, jnp.float32)


def _tc_finish(x_ref, thr_ref, svk_ref, out_ref, *, k_glob):
    sv = svk_ref[...]

    def count_ge(fc):
        return jnp.sum((sv >= fc).astype(jnp.int32))

    g0 = count_ge(jnp.float32(0.0))
    ans = jnp.where(g0 >= k_glob, jnp.int32(0), jnp.int32(-(2**31)))

    def body(i, ans):
        bit = jnp.int32(2**30) >> i
        cand = ans | bit
        cnt = count_ge(_key_to_float(cand))
        return jnp.where(cnt >= k_glob, cand, ans)

    ans = lax.fori_loop(0, 31, body, ans)
    tg = _key_to_float(ans)

    x = x_ref[...]
    thr = thr_ref[...]
    out_ref[...] = jnp.where((x >= thr) & (x != 0.0) & (x >= tg), x, 0.0)


def kernel(x):
    b, n = x.shape
    k_row = math.ceil(_K * n)
    k_glob = math.ceil(_K_PERCENT * b * k_row)
    rows_per_w = b // _NW
    cap_row = ((k_row + 25 + _L - 1) // _L) * _L

    mesh = plsc.VectorSubcoreMesh(core_axis_name="c", subcore_axis_name="s")
    sc = pl.kernel(
        functools.partial(_sc_body, k_row, rows_per_w, cap_row),
        out_type=(
            jax.ShapeDtypeStruct((_NW, rows_per_w * _L), jnp.float32),
            jax.ShapeDtypeStruct((_NW, rows_per_w * cap_row), jnp.float32),
        ),
        mesh=mesh,
        compiler_params=pltpu.CompilerParams(needs_layout_passes=False),
        scratch_types=[
            pltpu.VMEM((n,), jnp.float32),
            pltpu.VMEM((n // _L + _L,), jnp.int32),
            pltpu.VMEM((16 * 256,), jnp.int32),
            pltpu.VMEM((256 + _L,), jnp.int32),
            pltpu.VMEM((rows_per_w * _L,), jnp.float32),
            pltpu.VMEM((rows_per_w * cap_row,), jnp.float32),
        ],
    )
    thr_out, svk_out = sc(x)
    thr = thr_out.reshape(_NW, rows_per_w, _L)[:, :, 0].reshape(b, 1)

    return pl.pallas_call(
        functools.partial(_tc_finish, k_glob=k_glob),
        out_shape=jax.ShapeDtypeStruct((b, n), x.dtype),
        in_specs=[
            pl.BlockSpec(memory_space=pltpu.VMEM),
            pl.BlockSpec(memory_space=pltpu.VMEM),
            pl.BlockSpec(memory_space=pltpu.VMEM),
        ],
        out_specs=pl.BlockSpec(memory_space=pltpu.VMEM),
    )(x, thr, svk_out)
